# baseline (speedup 1.0000x reference)
import functools

import jax
import jax.numpy as jnp
import numpy as np
from jax.experimental import pallas as pl
from jax.experimental.pallas import tpu as pltpu
from jax.experimental.shard_map import shard_map

_TB = 256


def _conv1_indices():
    i = np.arange(12)[:, None, None]
    rho, s, c = i // 6, (i % 6) // 3, i % 3
    r = np.arange(128)[None, :, None]
    q, w = r // 32, r % 32
    col = np.arange(336)[None, None, :]
    g, u = col // 84, col % 84
    p, delta = g // 2, (g % 2) * 2
    j, o = u // 6, u % 6
    ki = 4 * s + q - (rho + delta)
    kw = w - (2 * j + p)
    mask = (ki >= 0) & (ki < 5) & (kw >= 0) & (kw < 5)
    idx = ((o * 3 + c) * 5 + np.clip(ki, 0, 4)) * 5 + np.clip(kw, 0, 4)
    return idx, mask


def _conv2_indices():
    i = np.arange(6)[:, None, None]
    rho, s = i // 3, i % 3
    r = np.arange(168)[None, :, None]
    half, u1 = r // 84, r % 84
    j1, ci = u1 // 6, u1 % 6
    col = np.arange(160)[None, None, :]
    p2, u2 = col // 80, col % 80
    j2, o2 = u2 // 16, u2 % 16
    ki = 2 * s + half - rho
    kw = j1 - (2 * j2 + p2)
    mask = (ki >= 0) & (ki < 5) & (kw >= 0) & (kw < 5)
    idx = ((o2 * 6 + ci) * 5 + np.clip(ki, 0, 4)) * 5 + np.clip(kw, 0, 4)
    return idx, mask


def _banded_conv_weights(w, w_in):
    cout, cin, kh, kw = w.shape
    w_out = w_in - kw + 1
    half = w_out // 2
    win = jnp.arange(w_in)[:, None]
    halves = []
    for parity in (0, 1):
        j2 = 2 * jnp.arange(half)[None, :] + parity
        kwi = win - j2
        valid = (kwi >= 0) & (kwi < kw)
        g = w[:, :, :, jnp.clip(kwi, 0, kw - 1)]
        g = jnp.where(valid[None, None, None], g, 0.0)
        g = jnp.transpose(g, (2, 3, 1, 4, 0))
        halves.append(g.reshape(kh, w_in * cin, half * cout))
    return jnp.concatenate(halves, axis=2)


def _conv1_weights(conv1_w):
    band = _banded_conv_weights(conv1_w, 32)
    z = jnp.zeros((32, 84), jnp.float32)

    def qblock(half, c, r, s):
        blocks = []
        for q in range(4):
            ki = 4 * s + q - r
            blocks.append(band[ki, c::3, 84 * half:84 * (half + 1)]
                          if 0 <= ki <= 4 else z)
        return jnp.concatenate(blocks, axis=0)

    ws = []
    for rho in (0, 1):
        for s in (0, 1):
            for c in range(3):
                ws.append(jnp.concatenate(
                    [qblock(0, c, rho, s), qblock(0, c, rho + 2, s),
                     qblock(1, c, rho, s), qblock(1, c, rho + 2, s)],
                    axis=1))
    return jnp.stack(ws)


def _conv2_weights(conv2_w):
    band = _banded_conv_weights(conv2_w, 14)
    z = jnp.zeros((84, 160), jnp.float32)

    def tap(ki):
        return band[ki] if 0 <= ki <= 4 else z

    ws = []
    for rho in (0, 1):
        for s in range(3):
            k0 = 2 * s - rho
            ws.append(jnp.concatenate([tap(k0), tap(k0 + 1)], axis=0))
    return jnp.stack(ws)


def _lenet_body(x0_ref, x1_ref, x2_ref, w1_ref, b1_ref, w2_ref, b2_ref,
                wf1_ref, bf1_ref, wf2_ref, bf2_ref, wf3_ref, bf3_ref,
                out_ref):
    f32 = jnp.float32
    xc = [r[...].reshape(_TB * 8, 128) for r in (x0_ref, x1_ref, x2_ref)]

    l1 = _TB * 8 - 1
    u = None
    v = None
    for s in (0, 1):
        for c in range(3):
            xs = xc[c][s:s + l1]
            du = jnp.dot(xs, w1_ref[s * 3 + c], preferred_element_type=f32)
            dv = jnp.dot(xs, w1_ref[6 + s * 3 + c], preferred_element_type=f32)
            u = du if u is None else u + du
            v = dv if v is None else v + dv
    w = jnp.maximum(u, v)
    pre = jnp.maximum(w[:, :168], w[:, 168:])
    p1 = jnp.maximum(pre + b1_ref[...], 0.0)

    l2 = l1 - 2
    u = jnp.dot(p1[0:l2], w2_ref[0], preferred_element_type=f32)
    v = jnp.dot(p1[0:l2], w2_ref[3], preferred_element_type=f32)
    for s in (1, 2):
        u = u + jnp.dot(p1[s:s + l2], w2_ref[s], preferred_element_type=f32)
        v = v + jnp.dot(p1[s:s + l2], w2_ref[3 + s], preferred_element_type=f32)
    pre = jnp.maximum(u, v)
    pre = jnp.maximum(pre[:, :80], pre[:, 80:])
    p2 = jnp.maximum(pre + b2_ref[...], 0.0)

    l3 = l2 - 4
    acc = jnp.dot(p2[0:l3], wf1_ref[0], preferred_element_type=f32)
    for h in range(1, 5):
        acc = acc + jnp.dot(p2[h:h + l3], wf1_ref[h],
                            preferred_element_type=f32)
    f1 = jnp.maximum(acc + bf1_ref[...], 0.0)

    f2 = jnp.maximum(jnp.dot(f1, wf2_ref[...], preferred_element_type=f32)
                     + bf2_ref[...], 0.0)
    f3 = (jnp.dot(f2, wf3_ref[...], preferred_element_type=f32)
          + bf3_ref[...])
    out_ref[...] = jnp.concatenate(
        [f3, jnp.zeros((_TB * 8 - l3, 10), f32)], axis=0)


@jax.jit
def _forward(conv1_w, conv1_b, conv2_w, conv2_b, fc1_w, fc1_b,
             fc2_w, fc2_b, fc3_w, fc3_b, x):
    n = x.shape[0]
    n_pad = -(-n // _TB) * _TB
    xr = x.astype(jnp.float32)
    if n_pad != n:
        xr = jnp.pad(xr, ((0, n_pad - n), (0, 0), (0, 0), (0, 0)))
    xq = xr.reshape(n_pad, 3, 8, 128)

    w1 = _conv1_weights(conv1_w)
    b1 = jnp.tile(conv1_b, 28)[None, :]
    w2 = _conv2_weights(conv2_w)
    b2 = jnp.tile(conv2_b, 5)[None, :]
    wf1 = fc1_w.reshape(120, 16, 5, 5).transpose(2, 3, 1, 0).reshape(5, 80, 120)
    bf1 = fc1_b[None, :]
    wf2 = fc2_w.T
    bf2 = fc2_b[None, :]
    wf3 = fc3_w.T
    bf3 = fc3_b[None, :]

    def w3(shape):
        return pl.BlockSpec(shape, lambda b: (0, 0, 0))

    def w2d(shape):
        return pl.BlockSpec(shape, lambda b: (0, 0))

    out = pl.pallas_call(
        _lenet_body,
        out_shape=jax.ShapeDtypeStruct((n_pad * 8, 10), jnp.float32),
        grid=(n_pad // _TB,),
        in_specs=[
            pl.BlockSpec((_TB, 1, 8, 128), lambda b: (b, 0, 0, 0)),
            pl.BlockSpec((_TB, 1, 8, 128), lambda b: (b, 1, 0, 0)),
            pl.BlockSpec((_TB, 1, 8, 128), lambda b: (b, 2, 0, 0)),
            w3((12, 128, 336)), w2d((1, 168)),
            w3((6, 168, 160)), w2d((1, 80)),
            w3((5, 80, 120)), w2d((1, 120)),
            w2d((120, 84)), w2d((1, 84)),
            w2d((84, 10)), w2d((1, 10)),
        ],
        out_specs=pl.BlockSpec((_TB * 8, 10), lambda b: (b, 0)),
        compiler_params=pltpu.CompilerParams(
            dimension_semantics=("parallel",)),
    )(xq, xq, xq, w1, b1, w2, b2, wf1, bf1, wf2, bf2, wf3, bf3)
    return out.reshape(n_pad, 8, 10)[:n, 0, :]


@functools.lru_cache(maxsize=None)
def _make_forward(devs, n):
    if len(devs) < 2 or n % 2:
        return _forward
    mesh = jax.sharding.Mesh(devs, ("d",))
    rep = jax.sharding.PartitionSpec()
    shard = jax.sharding.PartitionSpec("d")
    fwd = shard_map(_forward, mesh=mesh,
                    in_specs=(rep,) * 10 + (shard,), out_specs=shard,
                    check_rep=False)
    return jax.jit(fwd)


def kernel(conv1_w, conv1_b, conv2_w, conv2_b, fc1_w, fc1_b,
           fc2_w, fc2_b, fc3_w, fc3_b, x):
    devs = tuple(d for d in jax.devices() if d.platform != "cpu")[:2]
    fwd = _make_forward(devs, x.shape[0])
    return fwd(conv1_w, conv1_b, conv2_w, conv2_b, fc1_w, fc1_b,
               fc2_w, fc2_b, fc3_w, fc3_b, x)


# speedup vs baseline: 5.2216x; 5.2216x over previous
import jax
import jax.numpy as jnp
import numpy as np
from jax.experimental import pallas as pl
from jax.experimental.pallas import tpu as pltpu

_TB = 256


def _conv1_indices():
    i = np.arange(12)[:, None, None]
    rho, s, c = i // 6, (i % 6) // 3, i % 3
    r = np.arange(128)[None, :, None]
    q, w = r // 32, r % 32
    col = np.arange(336)[None, None, :]
    g, u = col // 84, col % 84
    p, delta = g // 2, (g % 2) * 2
    j, o = u // 6, u % 6
    ki = 4 * s + q - (rho + delta)
    kw = w - (2 * j + p)
    mask = (ki >= 0) & (ki < 5) & (kw >= 0) & (kw < 5)
    idx = ((o * 3 + c) * 5 + np.clip(ki, 0, 4)) * 5 + np.clip(kw, 0, 4)
    return idx, mask


def _conv2_indices():
    i = np.arange(6)[:, None, None]
    rho, s = i // 3, i % 3
    r = np.arange(168)[None, :, None]
    half, u1 = r // 84, r % 84
    j1, ci = u1 // 6, u1 % 6
    col = np.arange(160)[None, None, :]
    p2, u2 = col // 80, col % 80
    j2, o2 = u2 // 16, u2 % 16
    ki = 2 * s + half - rho
    kw = j1 - (2 * j2 + p2)
    mask = (ki >= 0) & (ki < 5) & (kw >= 0) & (kw < 5)
    idx = ((o2 * 6 + ci) * 5 + np.clip(ki, 0, 4)) * 5 + np.clip(kw, 0, 4)
    return idx, mask


def _banded_conv_weights(w, w_in):
    cout, cin, kh, kw = w.shape
    w_out = w_in - kw + 1
    half = w_out // 2
    win = jnp.arange(w_in)[:, None]
    halves = []
    for parity in (0, 1):
        j2 = 2 * jnp.arange(half)[None, :] + parity
        kwi = win - j2
        valid = (kwi >= 0) & (kwi < kw)
        g = w[:, :, :, jnp.clip(kwi, 0, kw - 1)]
        g = jnp.where(valid[None, None, None], g, 0.0)
        g = jnp.transpose(g, (2, 3, 1, 4, 0))
        halves.append(g.reshape(kh, w_in * cin, half * cout))
    return jnp.concatenate(halves, axis=2)


def _conv1_weights(conv1_w):
    band = _banded_conv_weights(conv1_w, 32)
    z = jnp.zeros((32, 84), jnp.float32)

    def qblock(half, c, r, s):
        blocks = []
        for q in range(4):
            ki = 4 * s + q - r
            blocks.append(band[ki, c::3, 84 * half:84 * (half + 1)]
                          if 0 <= ki <= 4 else z)
        return jnp.concatenate(blocks, axis=0)

    ws = []
    for rho in (0, 1):
        for s in (0, 1):
            for c in range(3):
                ws.append(jnp.concatenate(
                    [qblock(0, c, rho, s), qblock(0, c, rho + 2, s),
                     qblock(1, c, rho, s), qblock(1, c, rho + 2, s)],
                    axis=1))
    return jnp.stack(ws)


def _conv2_weights(conv2_w):
    band = _banded_conv_weights(conv2_w, 14)
    z = jnp.zeros((84, 160), jnp.float32)

    def tap(ki):
        return band[ki] if 0 <= ki <= 4 else z

    ws = []
    for rho in (0, 1):
        for s in range(3):
            k0 = 2 * s - rho
            ws.append(jnp.concatenate([tap(k0), tap(k0 + 1)], axis=0))
    return jnp.stack(ws)


def _lenet_body(x0_ref, x1_ref, x2_ref, w1_ref, b1_ref, w2_ref, b2_ref,
                wf1_ref, bf1_ref, wf2_ref, bf2_ref, wf3_ref, bf3_ref,
                out_ref):
    f32 = jnp.float32
    bf16 = jnp.bfloat16
    xc = [r[...].reshape(_TB * 8, 128).astype(bf16)
          for r in (x0_ref, x1_ref, x2_ref)]

    l1 = _TB * 8 - 1
    u = None
    v = None
    for s in (0, 1):
        for c in range(3):
            xs = xc[c][s:s + l1]
            du = jnp.dot(xs, w1_ref[s * 3 + c], preferred_element_type=f32)
            dv = jnp.dot(xs, w1_ref[6 + s * 3 + c], preferred_element_type=f32)
            u = du if u is None else u + du
            v = dv if v is None else v + dv
    w = jnp.maximum(u, v)
    pre = jnp.maximum(w[:, :168], w[:, 168:])
    p1 = jnp.maximum(pre + b1_ref[...], 0.0).astype(bf16)

    l2 = l1 - 2
    u = jnp.dot(p1[0:l2], w2_ref[0], preferred_element_type=f32)
    v = jnp.dot(p1[0:l2], w2_ref[3], preferred_element_type=f32)
    for s in (1, 2):
        u = u + jnp.dot(p1[s:s + l2], w2_ref[s], preferred_element_type=f32)
        v = v + jnp.dot(p1[s:s + l2], w2_ref[3 + s], preferred_element_type=f32)
    pre = jnp.maximum(u, v)
    pre = jnp.maximum(pre[:, :80], pre[:, 80:])
    p2 = jnp.maximum(pre + b2_ref[...], 0.0).astype(bf16)

    l3 = l2 - 4
    acc = jnp.dot(p2[0:l3], wf1_ref[0], preferred_element_type=f32)
    for h in range(1, 5):
        acc = acc + jnp.dot(p2[h:h + l3], wf1_ref[h],
                            preferred_element_type=f32)
    f1 = jnp.maximum(acc + bf1_ref[...], 0.0).astype(bf16)

    f2 = jnp.maximum(jnp.dot(f1, wf2_ref[...], preferred_element_type=f32)
                     + bf2_ref[...], 0.0).astype(bf16)
    f3 = (jnp.dot(f2, wf3_ref[...], preferred_element_type=f32)
          + bf3_ref[...])
    out_ref[...] = jnp.concatenate(
        [f3, jnp.zeros((_TB * 8 - l3, 10), f32)], axis=0)


@jax.jit
def _forward(conv1_w, conv1_b, conv2_w, conv2_b, fc1_w, fc1_b,
             fc2_w, fc2_b, fc3_w, fc3_b, x):
    n = x.shape[0]
    n_pad = -(-n // _TB) * _TB
    xr = x.astype(jnp.float32)
    if n_pad != n:
        xr = jnp.pad(xr, ((0, n_pad - n), (0, 0), (0, 0), (0, 0)))
    xq = xr.reshape(n_pad, 3, 8, 128)

    w1 = _conv1_weights(conv1_w).astype(jnp.bfloat16)
    b1 = jnp.tile(conv1_b, 28)[None, :]
    w2 = _conv2_weights(conv2_w).astype(jnp.bfloat16)
    b2 = jnp.tile(conv2_b, 5)[None, :]
    wf1 = fc1_w.reshape(120, 16, 5, 5).transpose(2, 3, 1, 0).reshape(
        5, 80, 120).astype(jnp.bfloat16)
    bf1 = fc1_b[None, :]
    wf2 = fc2_w.T.astype(jnp.bfloat16)
    bf2 = fc2_b[None, :]
    wf3 = fc3_w.T.astype(jnp.bfloat16)
    bf3 = fc3_b[None, :]

    def w3(shape):
        return pl.BlockSpec(shape, lambda b: (0, 0, 0))

    def w2d(shape):
        return pl.BlockSpec(shape, lambda b: (0, 0))

    out = pl.pallas_call(
        _lenet_body,
        out_shape=jax.ShapeDtypeStruct((n_pad * 8, 10), jnp.float32),
        grid=(n_pad // _TB,),
        in_specs=[
            pl.BlockSpec((_TB, 1, 8, 128), lambda b: (b, 0, 0, 0)),
            pl.BlockSpec((_TB, 1, 8, 128), lambda b: (b, 1, 0, 0)),
            pl.BlockSpec((_TB, 1, 8, 128), lambda b: (b, 2, 0, 0)),
            w3((12, 128, 336)), w2d((1, 168)),
            w3((6, 168, 160)), w2d((1, 80)),
            w3((5, 80, 120)), w2d((1, 120)),
            w2d((120, 84)), w2d((1, 84)),
            w2d((84, 10)), w2d((1, 10)),
        ],
        out_specs=pl.BlockSpec((_TB * 8, 10), lambda b: (b, 0)),
        compiler_params=pltpu.CompilerParams(
            dimension_semantics=("parallel",)),
    )(xq, xq, xq, w1, b1, w2, b2, wf1, bf1, wf2, bf2, wf3, bf3)
    return out.reshape(n_pad, 8, 10)[:n, 0, :]


def kernel(conv1_w, conv1_b, conv2_w, conv2_b, fc1_w, fc1_b,
           fc2_w, fc2_b, fc3_w, fc3_b, x):
    return _forward(conv1_w, conv1_b, conv2_w, conv2_b, fc1_w, fc1_b,
                    fc2_w, fc2_b, fc3_w, fc3_b, x)


# speedup vs baseline: 5.4310x; 1.0401x over previous
import jax
import jax.numpy as jnp
import numpy as np
from jax.experimental import pallas as pl
from jax.experimental.pallas import tpu as pltpu

_TB = 256


def _conv1_indices():
    i = np.arange(12)[:, None, None]
    rho, s, c = i // 6, (i % 6) // 3, i % 3
    r = np.arange(128)[None, :, None]
    q, w = r // 32, r % 32
    col = np.arange(336)[None, None, :]
    g, u = col // 84, col % 84
    p, delta = g // 2, (g % 2) * 2
    j, o = u // 6, u % 6
    ki = 4 * s + q - (rho + delta)
    kw = w - (2 * j + p)
    mask = (ki >= 0) & (ki < 5) & (kw >= 0) & (kw < 5)
    idx = ((o * 3 + c) * 5 + np.clip(ki, 0, 4)) * 5 + np.clip(kw, 0, 4)
    return idx, mask


def _conv2_indices():
    i = np.arange(6)[:, None, None]
    rho, s = i // 3, i % 3
    r = np.arange(168)[None, :, None]
    half, u1 = r // 84, r % 84
    j1, ci = u1 // 6, u1 % 6
    col = np.arange(160)[None, None, :]
    p2, u2 = col // 80, col % 80
    j2, o2 = u2 // 16, u2 % 16
    ki = 2 * s + half - rho
    kw = j1 - (2 * j2 + p2)
    mask = (ki >= 0) & (ki < 5) & (kw >= 0) & (kw < 5)
    idx = ((o2 * 6 + ci) * 5 + np.clip(ki, 0, 4)) * 5 + np.clip(kw, 0, 4)
    return idx, mask


def _banded_conv_weights(w, w_in):
    cout, cin, kh, kw = w.shape
    w_out = w_in - kw + 1
    half = w_out // 2
    win = jnp.arange(w_in)[:, None]
    halves = []
    for parity in (0, 1):
        j2 = 2 * jnp.arange(half)[None, :] + parity
        kwi = win - j2
        valid = (kwi >= 0) & (kwi < kw)
        g = w[:, :, :, jnp.clip(kwi, 0, kw - 1)]
        g = jnp.where(valid[None, None, None], g, 0.0)
        g = jnp.transpose(g, (2, 3, 1, 4, 0))
        halves.append(g.reshape(kh, w_in * cin, half * cout))
    return jnp.concatenate(halves, axis=2)


def _conv1_weights(conv1_w):
    band = _banded_conv_weights(conv1_w, 32)
    z = jnp.zeros((32, 84), jnp.float32)

    def qblock(half, c, r, s):
        blocks = []
        for q in range(4):
            ki = 4 * s + q - r
            blocks.append(band[ki, c::3, 84 * half:84 * (half + 1)]
                          if 0 <= ki <= 4 else z)
        return jnp.concatenate(blocks, axis=0)

    ws = []
    for rho in (0, 1):
        for s in (0, 1):
            for c in range(3):
                ws.append(jnp.concatenate(
                    [qblock(0, c, rho, s), qblock(0, c, rho + 2, s),
                     qblock(1, c, rho, s), qblock(1, c, rho + 2, s)],
                    axis=1))
    return jnp.stack(ws)


def _conv2_weights(conv2_w):
    band = _banded_conv_weights(conv2_w, 14)
    z = jnp.zeros((84, 160), jnp.float32)

    def tap(ki):
        return band[ki] if 0 <= ki <= 4 else z

    ws = []
    for rho in (0, 1):
        for s in range(3):
            k0 = 2 * s - rho
            ws.append(jnp.concatenate([tap(k0), tap(k0 + 1)], axis=0))
    return jnp.stack(ws)


def _lenet_body(x0_ref, x1_ref, x2_ref, w1_ref, b1_ref, w2_ref, b2_ref,
                wf1_ref, bf1_ref, wf2_ref, bf2_ref, wf3_ref, bf3_ref,
                out_ref):
    f32 = jnp.float32
    bf16 = jnp.bfloat16
    xc = [r[...].reshape(_TB * 8, 128).astype(bf16)
          for r in (x0_ref, x1_ref, x2_ref)]

    l1 = _TB * 8 - 1
    xcat = jnp.concatenate(
        [xc[0][0:l1], xc[1][0:l1], xc[2][0:l1],
         xc[0][1:1 + l1], xc[1][1:1 + l1], xc[2][1:1 + l1]], axis=1)
    uv = jnp.dot(xcat, w1_ref[...], preferred_element_type=f32)
    w = jnp.maximum(uv[:, :336], uv[:, 336:])
    pre = jnp.maximum(w[:, :168], w[:, 168:])
    p1 = jnp.maximum(pre + b1_ref[...], 0.0).astype(bf16)

    l2 = l1 - 2
    pcat = jnp.concatenate([p1[0:l2], p1[1:1 + l2], p1[2:2 + l2]], axis=1)
    uv = jnp.dot(pcat, w2_ref[...], preferred_element_type=f32)
    pre = jnp.maximum(uv[:, :160], uv[:, 160:])
    pre = jnp.maximum(pre[:, :80], pre[:, 80:])
    p2 = jnp.maximum(pre + b2_ref[...], 0.0).astype(bf16)

    l3 = l2 - 4
    fcat = jnp.concatenate([p2[h:h + l3] for h in range(5)], axis=1)
    acc = jnp.dot(fcat, wf1_ref[...], preferred_element_type=f32)
    f1 = jnp.maximum(acc + bf1_ref[...], 0.0).astype(bf16)

    f2 = jnp.maximum(jnp.dot(f1, wf2_ref[...], preferred_element_type=f32)
                     + bf2_ref[...], 0.0).astype(bf16)
    f3 = (jnp.dot(f2, wf3_ref[...], preferred_element_type=f32)
          + bf3_ref[...])
    out_ref[...] = jnp.concatenate(
        [f3, jnp.zeros((_TB * 8 - l3, 10), f32)], axis=0)


@jax.jit
def _forward(conv1_w, conv1_b, conv2_w, conv2_b, fc1_w, fc1_b,
             fc2_w, fc2_b, fc3_w, fc3_b, x):
    n = x.shape[0]
    n_pad = -(-n // _TB) * _TB
    xr = x.astype(jnp.float32)
    if n_pad != n:
        xr = jnp.pad(xr, ((0, n_pad - n), (0, 0), (0, 0), (0, 0)))
    xq = xr.reshape(n_pad, 3, 8, 128)

    w1s = _conv1_weights(conv1_w).astype(jnp.bfloat16)
    w1 = jnp.concatenate(
        [w1s[:6].reshape(768, 336),
         w1s[6:].reshape(768, 336)], axis=1)
    b1 = jnp.tile(conv1_b, 28)[None, :]
    w2s = _conv2_weights(conv2_w).astype(jnp.bfloat16)
    w2 = jnp.concatenate(
        [w2s[:3].reshape(504, 160),
         w2s[3:].reshape(504, 160)], axis=1)
    b2 = jnp.tile(conv2_b, 5)[None, :]
    wf1 = fc1_w.reshape(120, 16, 5, 5).transpose(2, 3, 1, 0).reshape(
        400, 120).astype(jnp.bfloat16)
    bf1 = fc1_b[None, :]
    wf2 = fc2_w.T.astype(jnp.bfloat16)
    bf2 = fc2_b[None, :]
    wf3 = fc3_w.T.astype(jnp.bfloat16)
    bf3 = fc3_b[None, :]

    def w3(shape):
        return pl.BlockSpec(shape, lambda b: (0, 0, 0))

    def w2d(shape):
        return pl.BlockSpec(shape, lambda b: (0, 0))

    out = pl.pallas_call(
        _lenet_body,
        out_shape=jax.ShapeDtypeStruct((n_pad * 8, 10), jnp.float32),
        grid=(n_pad // _TB,),
        in_specs=[
            pl.BlockSpec((_TB, 1, 8, 128), lambda b: (b, 0, 0, 0)),
            pl.BlockSpec((_TB, 1, 8, 128), lambda b: (b, 1, 0, 0)),
            pl.BlockSpec((_TB, 1, 8, 128), lambda b: (b, 2, 0, 0)),
            w2d((768, 672)), w2d((1, 168)),
            w2d((504, 320)), w2d((1, 80)),
            w2d((400, 120)), w2d((1, 120)),
            w2d((120, 84)), w2d((1, 84)),
            w2d((84, 10)), w2d((1, 10)),
        ],
        out_specs=pl.BlockSpec((_TB * 8, 10), lambda b: (b, 0)),
        compiler_params=pltpu.CompilerParams(
            dimension_semantics=("parallel",)),
    )(xq, xq, xq, w1, b1, w2, b2, wf1, bf1, wf2, bf2, wf3, bf3)
    return out.reshape(n_pad, 8, 10)[:n, 0, :]


def kernel(conv1_w, conv1_b, conv2_w, conv2_b, fc1_w, fc1_b,
           fc2_w, fc2_b, fc3_w, fc3_b, x):
    return _forward(conv1_w, conv1_b, conv2_w, conv2_b, fc1_w, fc1_b,
                    fc2_w, fc2_b, fc3_w, fc3_b, x)


# speedup vs baseline: 5.6864x; 1.0470x over previous
import jax
import jax.numpy as jnp
import numpy as np
from jax.experimental import pallas as pl
from jax.experimental.pallas import tpu as pltpu

_TB = 256


def _conv1_indices():
    i = np.arange(12)[:, None, None]
    rho, s, c = i // 6, (i % 6) // 3, i % 3
    r = np.arange(128)[None, :, None]
    q, w = r // 32, r % 32
    col = np.arange(336)[None, None, :]
    g, u = col // 84, col % 84
    p, delta = g // 2, (g % 2) * 2
    j, o = u // 6, u % 6
    ki = 4 * s + q - (rho + delta)
    kw = w - (2 * j + p)
    mask = (ki >= 0) & (ki < 5) & (kw >= 0) & (kw < 5)
    idx = ((o * 3 + c) * 5 + np.clip(ki, 0, 4)) * 5 + np.clip(kw, 0, 4)
    return idx, mask


def _conv2_indices():
    i = np.arange(6)[:, None, None]
    rho, s = i // 3, i % 3
    r = np.arange(168)[None, :, None]
    half, u1 = r // 84, r % 84
    j1, ci = u1 // 6, u1 % 6
    col = np.arange(160)[None, None, :]
    p2, u2 = col // 80, col % 80
    j2, o2 = u2 // 16, u2 % 16
    ki = 2 * s + half - rho
    kw = j1 - (2 * j2 + p2)
    mask = (ki >= 0) & (ki < 5) & (kw >= 0) & (kw < 5)
    idx = ((o2 * 6 + ci) * 5 + np.clip(ki, 0, 4)) * 5 + np.clip(kw, 0, 4)
    return idx, mask


def _banded_conv_weights(w, w_in):
    cout, cin, kh, kw = w.shape
    w_out = w_in - kw + 1
    half = w_out // 2
    win = jnp.arange(w_in)[:, None]
    halves = []
    for parity in (0, 1):
        j2 = 2 * jnp.arange(half)[None, :] + parity
        kwi = win - j2
        valid = (kwi >= 0) & (kwi < kw)
        g = w[:, :, :, jnp.clip(kwi, 0, kw - 1)]
        g = jnp.where(valid[None, None, None], g, 0.0)
        g = jnp.transpose(g, (2, 3, 1, 4, 0))
        halves.append(g.reshape(kh, w_in * cin, half * cout))
    return jnp.concatenate(halves, axis=2)


def _conv1_weights(conv1_w):
    band = _banded_conv_weights(conv1_w, 32)
    z = jnp.zeros((32, 84), jnp.float32)

    def qblock(half, c, r, s):
        blocks = []
        for q in range(4):
            ki = 4 * s + q - r
            blocks.append(band[ki, c::3, 84 * half:84 * (half + 1)]
                          if 0 <= ki <= 4 else z)
        return jnp.concatenate(blocks, axis=0)

    ws = []
    for rho in (0, 1):
        for s in (0, 1):
            for c in range(3):
                ws.append(jnp.concatenate(
                    [qblock(0, c, rho, s), qblock(0, c, rho + 2, s),
                     qblock(1, c, rho, s), qblock(1, c, rho + 2, s)],
                    axis=1))
    return jnp.stack(ws)


def _conv2_weights(conv2_w):
    band = _banded_conv_weights(conv2_w, 14)
    z = jnp.zeros((84, 160), jnp.float32)

    def tap(ki):
        return band[ki] if 0 <= ki <= 4 else z

    ws = []
    for rho in (0, 1):
        for s in range(3):
            k0 = 2 * s - rho
            ws.append(jnp.concatenate([tap(k0), tap(k0 + 1)], axis=0))
    return jnp.stack(ws)


def _lenet_body(x0_ref, x1_ref, x2_ref, w1_ref, b1_ref, w2_ref, b2_ref,
                wf1_ref, bf1_ref, wf2_ref, bf2_ref, wf3_ref, bf3_ref,
                out_ref):
    f32 = jnp.float32
    bf16 = jnp.bfloat16
    xc = [r[...].reshape(_TB * 8, 128).astype(bf16)
          for r in (x0_ref, x1_ref, x2_ref)]

    l1 = _TB * 8 - 1
    xcat = jnp.concatenate(
        [xc[0][0:l1], xc[1][0:l1], xc[2][0:l1],
         xc[0][1:1 + l1], xc[1][1:1 + l1], xc[2][1:1 + l1]], axis=1)
    uv = jnp.dot(xcat, w1_ref[...], preferred_element_type=f32)
    w = jnp.maximum(uv[:, :512], uv[:, 512:])
    pre = jnp.maximum(w[:, :256], w[:, 256:])
    p1 = jnp.maximum(pre + b1_ref[...], 0.0).astype(bf16)

    l2 = l1 - 2
    pcat = jnp.concatenate([p1[0:l2], p1[1:1 + l2], p1[2:2 + l2]], axis=1)
    uv = jnp.dot(pcat, w2_ref[...], preferred_element_type=f32)
    pre = jnp.maximum(uv[:, :256], uv[:, 256:])
    pre = jnp.maximum(pre[:, :128], pre[:, 128:])
    p2 = jnp.maximum(pre + b2_ref[...], 0.0).astype(bf16)

    l3 = l2 - 4
    fcat = jnp.concatenate([p2[h:h + l3] for h in range(5)], axis=1)
    acc = jnp.dot(fcat, wf1_ref[...], preferred_element_type=f32)
    f1 = jnp.maximum(acc + bf1_ref[...], 0.0).astype(bf16)

    f2 = jnp.maximum(jnp.dot(f1, wf2_ref[...], preferred_element_type=f32)
                     + bf2_ref[...], 0.0).astype(bf16)
    f3 = (jnp.dot(f2, wf3_ref[...], preferred_element_type=f32)
          + bf3_ref[...])
    out_ref[...] = jnp.concatenate(
        [f3, jnp.zeros((_TB * 8 - l3, 10), f32)], axis=0)


@jax.jit
def _forward(conv1_w, conv1_b, conv2_w, conv2_b, fc1_w, fc1_b,
             fc2_w, fc2_b, fc3_w, fc3_b, x):
    n = x.shape[0]
    n_pad = -(-n // _TB) * _TB
    xr = x.astype(jnp.float32)
    if n_pad != n:
        xr = jnp.pad(xr, ((0, n_pad - n), (0, 0), (0, 0), (0, 0)))
    xq = xr.reshape(n_pad, 3, 8, 128)

    w1s = _conv1_weights(conv1_w).astype(jnp.bfloat16)
    w1 = jnp.concatenate(
        [w1s[:6].reshape(768, 336),
         w1s[6:].reshape(768, 336)], axis=1)
    w1 = jnp.pad(w1.reshape(768, 8, 84), ((0, 0), (0, 0), (0, 44))
                 ).reshape(768, 1024)
    b1 = jnp.pad(jnp.tile(conv1_b, 28).reshape(2, 84),
                 ((0, 0), (0, 44))).reshape(1, 256)
    w2s = _conv2_weights(conv2_w).astype(jnp.bfloat16)
    w2 = jnp.concatenate(
        [w2s[:3].reshape(504, 160),
         w2s[3:].reshape(504, 160)], axis=1)
    w2 = jnp.pad(w2.reshape(504, 4, 80), ((0, 0), (0, 0), (0, 48))
                 ).reshape(6, 84, 512)
    w2 = jnp.pad(w2, ((0, 0), (0, 44), (0, 0))).reshape(768, 512)
    b2 = jnp.pad(jnp.tile(conv2_b, 5)[None, :], ((0, 0), (0, 48)))
    wf1 = fc1_w.reshape(120, 16, 5, 5).transpose(2, 3, 1, 0).reshape(
        5, 80, 120).astype(jnp.bfloat16)
    wf1 = jnp.pad(wf1, ((0, 0), (0, 48), (0, 0))).reshape(640, 120)
    bf1 = fc1_b[None, :]
    wf2 = fc2_w.T.astype(jnp.bfloat16)
    bf2 = fc2_b[None, :]
    wf3 = fc3_w.T.astype(jnp.bfloat16)
    bf3 = fc3_b[None, :]

    def w3(shape):
        return pl.BlockSpec(shape, lambda b: (0, 0, 0))

    def w2d(shape):
        return pl.BlockSpec(shape, lambda b: (0, 0))

    out = pl.pallas_call(
        _lenet_body,
        out_shape=jax.ShapeDtypeStruct((n_pad * 8, 10), jnp.float32),
        grid=(n_pad // _TB,),
        in_specs=[
            pl.BlockSpec((_TB, 1, 8, 128), lambda b: (b, 0, 0, 0)),
            pl.BlockSpec((_TB, 1, 8, 128), lambda b: (b, 1, 0, 0)),
            pl.BlockSpec((_TB, 1, 8, 128), lambda b: (b, 2, 0, 0)),
            w2d((768, 1024)), w2d((1, 256)),
            w2d((768, 512)), w2d((1, 128)),
            w2d((640, 120)), w2d((1, 120)),
            w2d((120, 84)), w2d((1, 84)),
            w2d((84, 10)), w2d((1, 10)),
        ],
        out_specs=pl.BlockSpec((_TB * 8, 10), lambda b: (b, 0)),
        compiler_params=pltpu.CompilerParams(
            dimension_semantics=("parallel",)),
    )(xq, xq, xq, w1, b1, w2, b2, wf1, bf1, wf2, bf2, wf3, bf3)
    return out.reshape(n_pad, 8, 10)[:n, 0, :]


def kernel(conv1_w, conv1_b, conv2_w, conv2_b, fc1_w, fc1_b,
           fc2_w, fc2_b, fc3_w, fc3_b, x):
    return _forward(conv1_w, conv1_b, conv2_w, conv2_b, fc1_w, fc1_b,
                    fc2_w, fc2_b, fc3_w, fc3_b, x)
